# baseline (speedup 1.0000x reference)
import functools

import jax
import jax.numpy as jnp
from jax.experimental import pallas as pl
from jax.experimental.pallas import tpu as pltpu

B, D, H, E, G, K = 2048, 1024, 1024, 8, 64, 2
H2 = H // 2


def _gating_body(x_ref, wg1_ref, bg1_ref, wg2_ref, bg2_ref, gates_ref):
    x = x_ref[...]
    h = jnp.maximum(
        jnp.dot(x, wg1_ref[...], preferred_element_type=jnp.float32) + bg1_ref[...],
        0.0,
    )
    logits = jnp.dot(h, wg2_ref[...], preferred_element_type=jnp.float32) + bg2_ref[...]
    m = jnp.max(logits, axis=-1, keepdims=True)
    p = jnp.exp(logits - m)
    gates = p / jnp.sum(p, axis=-1, keepdims=True)
    eidx = jax.lax.broadcasted_iota(jnp.int32, gates.shape, 1)
    m1 = jnp.max(gates, axis=-1, keepdims=True)
    i1 = jnp.min(jnp.where(gates == m1, eidx, E), axis=-1, keepdims=True)
    g_wo1 = jnp.where(eidx == i1, -1.0, gates)
    m2 = jnp.max(g_wo1, axis=-1, keepdims=True)
    i2 = jnp.min(jnp.where(g_wo1 == m2, eidx, E), axis=-1, keepdims=True)
    keep = (eidx == i1) | (eidx == i2)
    gk = jnp.where(keep, gates, 0.0)
    gates_ref[...] = gk / (jnp.sum(gk, axis=-1, keepdims=True) + 1e-10)


def _expert_body(gT_ref, x_ref, w1_ref, b1_ref, w2_ref, b2_ref, w3_ref,
                 b3_ref, out_ref):
    e = pl.program_id(0)
    x = x_ref[...]
    h1 = jnp.maximum(
        jnp.dot(x, w1_ref[0], preferred_element_type=jnp.float32) + b1_ref[0],
        0.0,
    )
    h2 = jnp.maximum(
        jnp.dot(h1, w2_ref[0], preferred_element_type=jnp.float32) + b2_ref[0],
        0.0,
    )
    z = jnp.sum(h2 * w3_ref[0], axis=-1) + b3_ref[e]
    o = 1.0 / (1.0 + jnp.exp(-z))
    contrib = o * gT_ref[0, 0]

    @pl.when(e == 0)
    def _():
        out_ref[...] = contrib

    @pl.when(e != 0)
    def _():
        out_ref[...] += contrib


def _moe(x, wg1, bg1, wg2, bg2, w1, b1, w2, b2, w3, b3, *, interpret=False):
    gates = pl.pallas_call(
        _gating_body,
        out_shape=jax.ShapeDtypeStruct((B, E), jnp.float32),
        interpret=interpret,
    )(x, wg1, bg1.reshape(1, G), wg2, bg2.reshape(1, E))

    gT = gates.T.reshape(E, 1, B)
    pred = pl.pallas_call(
        _expert_body,
        grid=(E,),
        in_specs=[
            pl.BlockSpec((1, 1, B), lambda e: (e, 0, 0)),
            pl.BlockSpec((B, D), lambda e: (0, 0)),
            pl.BlockSpec((1, D, H), lambda e: (e, 0, 0)),
            pl.BlockSpec((1, 1, H), lambda e: (e, 0, 0)),
            pl.BlockSpec((1, H, H2), lambda e: (e, 0, 0)),
            pl.BlockSpec((1, 1, H2), lambda e: (e, 0, 0)),
            pl.BlockSpec((1, 1, H2), lambda e: (e, 0, 0)),
            pl.BlockSpec(memory_space=pltpu.SMEM),
        ],
        out_specs=pl.BlockSpec((B,), lambda e: (0,)),
        out_shape=jax.ShapeDtypeStruct((B,), jnp.float32),
        interpret=interpret,
    )(
        gT,
        x,
        w1,
        b1.reshape(E, 1, H),
        w2,
        b2.reshape(E, 1, H2),
        w3.reshape(E, 1, H2),
        b3.reshape(E),
    )
    return pred.reshape(B, 1), gates


def kernel(inputs, Wg1, bg1, Wg2, bg2, W1, b1, W2, b2, W3, b3):
    return _moe(inputs, Wg1, bg1, Wg2, bg2, W1, b1, W2, b2, W3, b3)


# speedup vs baseline: 1.0098x; 1.0098x over previous
import functools

import jax
import jax.numpy as jnp
from jax.experimental import pallas as pl
from jax.experimental.pallas import tpu as pltpu

B, D, H, E, G, K = 2048, 1024, 1024, 8, 64, 2
H2 = H // 2


def _gating_body(x_ref, wg1_ref, bg1_ref, wg2_ref, bg2_ref, gates_ref):
    x = x_ref[...]
    h = jnp.maximum(
        jnp.dot(x, wg1_ref[...], preferred_element_type=jnp.float32) + bg1_ref[...],
        0.0,
    )
    logits = jnp.dot(h, wg2_ref[...], preferred_element_type=jnp.float32) + bg2_ref[...]
    m = jnp.max(logits, axis=-1, keepdims=True)
    p = jnp.exp(logits - m)
    gates = p / jnp.sum(p, axis=-1, keepdims=True)
    eidx = jax.lax.broadcasted_iota(jnp.int32, gates.shape, 1)
    m1 = jnp.max(gates, axis=-1, keepdims=True)
    i1 = jnp.min(jnp.where(gates == m1, eidx, E), axis=-1, keepdims=True)
    g_wo1 = jnp.where(eidx == i1, -1.0, gates)
    m2 = jnp.max(g_wo1, axis=-1, keepdims=True)
    i2 = jnp.min(jnp.where(g_wo1 == m2, eidx, E), axis=-1, keepdims=True)
    keep = (eidx == i1) | (eidx == i2)
    gk = jnp.where(keep, gates, 0.0)
    gates_ref[...] = gk / (jnp.sum(gk, axis=-1, keepdims=True) + 1e-10)


def _expert_body(gT_ref, x_ref, w1_ref, b1_ref, w2_ref, b2_ref, w3_ref,
                 b3_ref, out_ref):
    e = pl.program_id(0)
    x = x_ref[...].astype(jnp.bfloat16)
    h1 = jnp.maximum(
        jnp.dot(x, w1_ref[0].astype(jnp.bfloat16),
                preferred_element_type=jnp.float32) + b1_ref[0],
        0.0,
    ).astype(jnp.bfloat16)
    h2 = jnp.maximum(
        jnp.dot(h1, w2_ref[0].astype(jnp.bfloat16),
                preferred_element_type=jnp.float32) + b2_ref[0],
        0.0,
    )
    z = jnp.sum(h2 * w3_ref[0], axis=-1) + b3_ref[e]
    o = 1.0 / (1.0 + jnp.exp(-z))
    contrib = o * gT_ref[0, 0]

    @pl.when(e == 0)
    def _():
        out_ref[...] = contrib

    @pl.when(e != 0)
    def _():
        out_ref[...] += contrib


def _moe(x, wg1, bg1, wg2, bg2, w1, b1, w2, b2, w3, b3, *, interpret=False):
    gates = pl.pallas_call(
        _gating_body,
        out_shape=jax.ShapeDtypeStruct((B, E), jnp.float32),
        interpret=interpret,
    )(x, wg1, bg1.reshape(1, G), wg2, bg2.reshape(1, E))

    gT = gates.T.reshape(E, 1, B)
    pred = pl.pallas_call(
        _expert_body,
        grid=(E,),
        in_specs=[
            pl.BlockSpec((1, 1, B), lambda e: (e, 0, 0)),
            pl.BlockSpec((B, D), lambda e: (0, 0)),
            pl.BlockSpec((1, D, H), lambda e: (e, 0, 0)),
            pl.BlockSpec((1, 1, H), lambda e: (e, 0, 0)),
            pl.BlockSpec((1, H, H2), lambda e: (e, 0, 0)),
            pl.BlockSpec((1, 1, H2), lambda e: (e, 0, 0)),
            pl.BlockSpec((1, 1, H2), lambda e: (e, 0, 0)),
            pl.BlockSpec(memory_space=pltpu.SMEM),
        ],
        out_specs=pl.BlockSpec((B,), lambda e: (0,)),
        out_shape=jax.ShapeDtypeStruct((B,), jnp.float32),
        interpret=interpret,
    )(
        gT,
        x,
        w1,
        b1.reshape(E, 1, H),
        w2,
        b2.reshape(E, 1, H2),
        w3.reshape(E, 1, H2),
        b3.reshape(E),
    )
    return pred.reshape(B, 1), gates


def kernel(inputs, Wg1, bg1, Wg2, bg2, W1, b1, W2, b2, W3, b3):
    return _moe(inputs, Wg1, bg1, Wg2, bg2, W1, b1, W2, b2, W3, b3)
